# baseline (speedup 1.0000x reference)
import functools
import math

import jax
import jax.numpy as jnp
from jax import lax
from jax.experimental import pallas as pl
from jax.experimental.pallas import tpu as pltpu
from jax.experimental.pallas import tpu_sc as plsc

_TS = 128
_NBUF = 4


def _build_emb(n_tok, d, scale):
    nc, ns = 2, 16
    n_workers = nc * ns
    toks_per_w = n_tok // n_workers
    n_steps = toks_per_w // _TS
    assert n_steps % _NBUF == 0
    d2 = 2 * d
    mesh = plsc.VectorSubcoreMesh(core_axis_name="c", subcore_axis_name="s")

    @functools.partial(
        pl.kernel,
        mesh=mesh,
        out_type=jax.ShapeDtypeStruct((n_tok // 2, d2), jnp.float32),
        scratch_types=[
            pltpu.VMEM((n_steps, _TS), jnp.int32),
            pltpu.VMEM((_NBUF, _TS, d2), jnp.float32),
            pltpu.VMEM((_NBUF, _TS // 2, d2), jnp.float32),
            pltpu.VMEM((_NBUF, _TS), jnp.int32),
            pltpu.SemaphoreType.DMA((_NBUF,)),
            pltpu.SemaphoreType.DMA((_NBUF,)),
        ],
        compiler_params=pltpu.CompilerParams(use_tc_tiling_on_sc=False),
    )
    def emb(idx_hbm, tab2_hbm, out_hbm, idx_v, rows_g, out_s, pid_v, gsem,
            ssem):
        wid = lax.axis_index("s") * nc + lax.axis_index("c")
        base_step = wid * n_steps
        pltpu.sync_copy(idx_hbm.at[pl.ds(base_step, n_steps)], idx_v)

        def start_gather(step, b):
            def mk(c, carry):
                sl = pl.ds(c * 16, 16)
                pid_v[b, sl] = lax.shift_right_logical(idx_v[step, sl], 1)
                return carry

            lax.fori_loop(0, _TS // 16, mk, 0)
            pltpu.async_copy(
                tab2_hbm.at[pid_v.at[b]], rows_g.at[b], gsem.at[b]
            )

        for b in range(_NBUF):
            start_gather(b, b)

        def group(g, carry):
            for b in range(_NBUF):
                step = g * _NBUF + b
                pltpu.make_async_copy(
                    tab2_hbm.at[pid_v.at[b]], rows_g.at[b], gsem.at[b]
                ).wait()

                @pl.when(step >= _NBUF)
                def _():
                    pltpu.make_async_copy(
                        out_s.at[b],
                        out_hbm.at[pl.ds(0, _TS // 2)],
                        ssem.at[b],
                    ).wait()

                def chunk_body(c, c2):
                    tvec = idx_v[step, pl.ds(c * 16, 16)]
                    for i in range(16):
                        t = tvec[i]
                        h = lax.mul(lax.rem(t, 2), d)
                        j = c * 16 + i
                        jj = c * 8 + i // 2
                        o = (i % 2) * d
                        for k in range(d // 16):
                            v = rows_g[b, j, pl.ds(h + k * 16, 16)] * scale
                            out_s[b, jj, pl.ds(o + k * 16, 16)] = v
                    return c2

                lax.fori_loop(0, _TS // 16, chunk_body, 0)
                pltpu.async_copy(
                    out_s.at[b],
                    out_hbm.at[
                        pl.ds((base_step + step) * (_TS // 2), _TS // 2)
                    ],
                    ssem.at[b],
                )

                @pl.when(step + _NBUF < n_steps)
                def _():
                    start_gather(step + _NBUF, b)

            return carry

        lax.fori_loop(0, n_steps // _NBUF, group, 0)

        for b in range(_NBUF):
            pltpu.make_async_copy(
                out_s.at[b], out_hbm.at[pl.ds(0, _TS // 2)], ssem.at[b]
            ).wait()

    return emb


def kernel(tokens, table):
    n_b, n_l = tokens.shape
    v, d = table.shape
    n = n_b * n_l
    scale = math.sqrt(d)
    idx = tokens.reshape(n // _TS, _TS)
    tab2 = table.reshape(v // 2, 2 * d)
    out2 = _build_emb(n, d, scale)(idx, tab2)
    return out2.reshape(n_b, n_l, d)


# speedup vs baseline: 1.5759x; 1.5759x over previous
import functools
import math

import jax
import jax.numpy as jnp
from jax import lax
from jax.experimental import pallas as pl
from jax.experimental.pallas import tpu as pltpu
from jax.experimental.pallas import tpu_sc as plsc

_TS = 128
_GBUF = 4
_SBUF = 2


def _build_emb(n_tok, d, d2, scale):
    nc, ns = 2, 16
    n_workers = nc * ns
    toks_per_w = n_tok // n_workers
    n_steps = toks_per_w // _TS
    assert n_steps % _GBUF == 0 and _GBUF % _SBUF == 0
    mesh = plsc.VectorSubcoreMesh(core_axis_name="c", subcore_axis_name="s")

    @functools.partial(
        pl.kernel,
        mesh=mesh,
        out_type=jax.ShapeDtypeStruct((n_tok, d2), jnp.float32),
        scratch_types=[
            pltpu.VMEM((n_steps, _TS), jnp.int32),
            pltpu.VMEM((_GBUF, _TS, d2), jnp.float32),
            pltpu.VMEM((_SBUF, _TS, d2), jnp.float32),
            pltpu.SemaphoreType.DMA((_GBUF,)),
            pltpu.SemaphoreType.DMA((_SBUF,)),
        ],
        compiler_params=pltpu.CompilerParams(use_tc_tiling_on_sc=False),
    )
    def emb(idx_hbm, tab_hbm, out_hbm, idx_v, rows_g, out_s, gsem, ssem):
        wid = lax.axis_index("s") * nc + lax.axis_index("c")
        base_step = wid * n_steps
        pltpu.sync_copy(idx_hbm.at[pl.ds(base_step, n_steps)], idx_v)

        def start_gather(step, b):
            pltpu.async_copy(
                tab_hbm.at[idx_v.at[step]], rows_g.at[b], gsem.at[b]
            )

        for b in range(_GBUF):
            start_gather(b, b)

        def group(g, carry):
            for b in range(_GBUF):
                step = g * _GBUF + b
                s = b % _SBUF
                pltpu.make_async_copy(
                    tab_hbm.at[idx_v.at[step]], rows_g.at[b], gsem.at[b]
                ).wait()

                @pl.when(step >= _SBUF)
                def _():
                    pltpu.make_async_copy(
                        out_s.at[s], out_hbm.at[pl.ds(0, _TS)], ssem.at[s]
                    ).wait()

                def scale_row(r, c2):
                    for k in range(d // 16):
                        sl = pl.ds(k * 16, 16)
                        out_s[s, r, sl] = rows_g[b, r, sl] * scale
                    return c2

                lax.fori_loop(0, _TS, scale_row, 0)
                pltpu.async_copy(
                    out_s.at[s],
                    out_hbm.at[pl.ds((base_step + step) * _TS, _TS)],
                    ssem.at[s],
                )

                @pl.when(step + _GBUF < n_steps)
                def _():
                    start_gather(step + _GBUF, b)

            return carry

        lax.fori_loop(0, n_steps // _GBUF, group, 0)

        for s in range(_SBUF):
            pltpu.make_async_copy(
                out_s.at[s], out_hbm.at[pl.ds(0, _TS)], ssem.at[s]
            ).wait()

    return emb


def kernel(tokens, table):
    n_b, n_l = tokens.shape
    v, d = table.shape
    n = n_b * n_l
    d2 = 2 * d
    scale = math.sqrt(d)
    idx = tokens.reshape(n // _TS, _TS)
    tab_pad = jnp.pad(table, ((0, 0), (0, d2 - d)))
    o = _build_emb(n, d, d2, scale)(idx, tab_pad)
    return o[:, :d].reshape(n_b, n_l, d)


# speedup vs baseline: 1.8203x; 1.1550x over previous
import functools
import math

import jax
import jax.numpy as jnp
from jax import lax
from jax.experimental import pallas as pl
from jax.experimental.pallas import tpu as pltpu
from jax.experimental.pallas import tpu_sc as plsc

_TS = 128
_NB = 6
_PF = 3
_TCB = 2048


def _tc_prep(v, d, d2, scale):
    grid = (v + _TCB - 1) // _TCB

    def body(t_ref, o_ref):
        o_ref[:, :d] = t_ref[...].T * scale

    return pl.pallas_call(
        body,
        grid=(grid,),
        in_specs=[pl.BlockSpec((d, _TCB), lambda i: (0, i))],
        out_specs=pl.BlockSpec((_TCB, d2), lambda i: (i, 0)),
        out_shape=jax.ShapeDtypeStruct((v, d2), jnp.float32),
    )


def _sc_gather(n_tok, d, d2):
    nc, ns = 2, 16
    n_workers = nc * ns
    n_steps = n_tok // n_workers // _TS
    assert _PF < _NB
    mesh = plsc.VectorSubcoreMesh(core_axis_name="c", subcore_axis_name="s")

    @functools.partial(
        pl.kernel,
        mesh=mesh,
        out_type=jax.ShapeDtypeStruct((n_tok, d2), jnp.float32),
        scratch_types=[
            pltpu.VMEM((n_steps, _TS), jnp.int32),
            pltpu.VMEM((_NB, _TS, d2), jnp.float32),
            pltpu.SemaphoreType.DMA((_NB,)),
            pltpu.SemaphoreType.DMA((_NB,)),
        ],
        compiler_params=pltpu.CompilerParams(use_tc_tiling_on_sc=False),
    )
    def emb(idx_hbm, tab_hbm, out_hbm, idx_v, rows, gsem, ssem):
        wid = lax.axis_index("s") * nc + lax.axis_index("c")
        base_step = wid * n_steps
        pltpu.sync_copy(idx_hbm.at[pl.ds(base_step, n_steps)], idx_v)

        def start_gather(step):
            b = step % _NB
            pltpu.async_copy(
                tab_hbm.at[idx_v.at[step]], rows.at[b], gsem.at[b]
            )

        def store_dst(step):
            return out_hbm.at[
                pl.ds((base_step + step) * _TS, _TS), pl.ds(0, d)
            ]

        for s in range(_PF):
            start_gather(s)

        def step_body(s, carry):
            b = s % _NB
            pltpu.make_async_copy(
                tab_hbm.at[idx_v.at[s]], rows.at[b], gsem.at[b]
            ).wait()
            pltpu.async_copy(
                rows.at[b, slice(None), pl.ds(0, d)], store_dst(s),
                ssem.at[b],
            )

            @pl.when(s + _PF >= _NB)
            def _():
                b2 = (s + _PF) % _NB
                pltpu.make_async_copy(
                    rows.at[b2, slice(None), pl.ds(0, d)],
                    store_dst(0),
                    ssem.at[b2],
                ).wait()

            @pl.when(s + _PF < n_steps)
            def _():
                start_gather(s + _PF)

            return carry

        lax.fori_loop(0, n_steps, step_body, 0)

        for s in range(n_steps - (_NB - _PF), n_steps):
            b = s % _NB
            pltpu.make_async_copy(
                rows.at[b, slice(None), pl.ds(0, d)], store_dst(0),
                ssem.at[b],
            ).wait()

    return emb


def kernel(tokens, table):
    n_b, n_l = tokens.shape
    v, d = table.shape
    n = n_b * n_l
    d2 = 2 * d
    scale = math.sqrt(d)
    tab_t = jnp.transpose(table)
    tab_rows = _tc_prep(v, d, d2, scale)(tab_t)
    idx = tokens.reshape(n // _TS, _TS)
    o = _sc_gather(n, d, d2)(idx, tab_rows)
    return o[:, :d].reshape(n_b, n_l, d)
